# baseline (speedup 1.0000x reference)
import jax
import jax.numpy as jnp
from jax.experimental import pallas as pl

N_ENT = 10000
N_REL = 474
E = 160000
H = 256


def _softmax_seg(logits, dst):
    m = jax.ops.segment_max(logits, dst, num_segments=N_ENT)
    e = jnp.exp(logits - m[dst])
    s = jax.ops.segment_sum(e, dst, num_segments=N_ENT)
    return e / (s[dst] + 1e-16)


def _add_kernel(x_ref, y_ref, o_ref):
    o_ref[...] = x_ref[...] + y_ref[...]


def _edge(ent, rel, W, src, dst, rel_id):
    e_emb = rel[rel_id]
    norm = jnp.sum(e_emb * ent[dst], axis=-1)
    a = _softmax_seg(norm, dst)
    neigh = jax.ops.segment_sum(e_emb * a[:, None], dst, num_segments=N_ENT)
    return jnp.tanh(neigh @ W)


def _node(ent, W, src, dst):
    norm = jnp.sum(ent[src] * ent[dst], axis=-1)
    a = _softmax_seg(norm, dst)
    neigh = jax.ops.segment_sum(ent[src] * a[:, None], dst, num_segments=N_ENT)
    return jnp.tanh(neigh @ W)


def _comp(ent, rel, W, src, dst, rel_id):
    comp = ent[src] * rel[rel_id]
    norm = jnp.sum(comp * ent[dst], axis=-1)
    a = _softmax_seg(norm, dst)
    neigh = jax.ops.segment_sum(comp * a[:, None], dst, num_segments=N_ENT)
    return jnp.tanh(neigh @ W)


def kernel(edge_index, rel_id, ent_emb, rel_emb_0, rel_emb_1,
           W_edge_0, W_node_0, W_comp_0, W_edge_1, W_node_1, W_comp_1):
    src = edge_index[0]
    dst = edge_index[1]
    ent = ent_emb
    for rel, We, Wn, Wc in ((rel_emb_0, W_edge_0, W_node_0, W_comp_0),
                            (rel_emb_1, W_edge_1, W_node_1, W_comp_1)):
        e_ent = _edge(ent, rel, We, src, dst, rel_id)
        n_ent = _node(ent, Wn, src, dst)
        c_ent = _comp(ent, rel, Wc, src, dst, rel_id)
        upd = pl.pallas_call(
            _add_kernel,
            out_shape=jax.ShapeDtypeStruct((N_ENT, H), jnp.float32),
        )(e_ent + n_ent, c_ent)
        ent = ent + upd
    return ent


# speedup vs baseline: 2.4206x; 2.4206x over previous
import jax
import jax.numpy as jnp
from jax import lax
from jax.experimental import pallas as pl
from jax.experimental.pallas import tpu as pltpu
from jax.experimental.pallas import tpu_sc as plsc

N = 10000
NREL = 474
E = 160000
H = 256

NC = 2
NS = 16
NW = NC * NS
C = 48
NCH = 105
EPW = C * NCH
EP = EPW * NW
NROW = EP // C
ACC_R = 10240
RPT = ACC_R // NS
QH = 32
NQ = H // QH
APAD = 10240
BN = 512
NEGB = -1000.0


def _lanes():
    return lax.iota(jnp.int32, 16)


def _lg(x, idx):
    dnums = lax.GatherDimensionNumbers(
        offset_dims=(), collapsed_slice_dims=(0,), start_index_map=(0,))
    return lax.gather(x, idx[:, None], dnums, (1,),
                      mode=lax.GatherScatterMode.PROMISE_IN_BOUNDS)


def _hsum(x):
    lane = _lanes()
    for d in (1, 2, 4, 8):
        x = x + _lg(x, lane ^ d)
    return x


def _k1_body(ent_h, rel_h, src_h, dstg_h, dsts_h, rid_h,
             n0_h, n1_h, n2_h, tacc_h,
             idx_s, idx_dg, idx_ds, idx_r, rows_s, rows_d, rows_r,
             nbuf, trow, zbuf16, tacc, sem0, sem1, sem2, sem3):
    cid = lax.axis_index("c")
    sid = lax.axis_index("s")
    wid = cid * NS + sid
    lane = _lanes()

    def zb(i, _):
        zbuf16[i, pl.ds(0, 16)] = jnp.zeros((16,), jnp.float32)
        return 0
    lax.fori_loop(0, RPT, zb, 0)
    pltpu.sync_copy(zbuf16, tacc.at[pl.ds(sid * RPT, RPT)])
    plsc.subcore_barrier()

    def chunk(ch, _):
        r = wid * NCH + ch
        pltpu.sync_copy(src_h.at[r], idx_s)
        pltpu.sync_copy(dstg_h.at[r], idx_dg)
        pltpu.sync_copy(dsts_h.at[r], idx_ds)
        pltpu.sync_copy(rid_h.at[r], idx_r)
        ca = pltpu.async_copy(ent_h.at[idx_s], rows_s, sem0)
        cb = pltpu.async_copy(ent_h.at[idx_dg], rows_d, sem1)
        cc = pltpu.async_copy(rel_h.at[idx_r], rows_r, sem2)
        ca.wait()
        cb.wait()
        cc.wait()

        def group(g, _):
            def edge(j16, carry):
                vn, vc, ve = carry
                j = g * 16 + j16
                accn = jnp.zeros((16,), jnp.float32)
                accc = jnp.zeros((16,), jnp.float32)
                acce = jnp.zeros((16,), jnp.float32)
                for hv in range(16):
                    s4 = rows_s[j, pl.ds(hv * 16, 16)]
                    d4 = rows_d[j, pl.ds(hv * 16, 16)]
                    r4 = rows_r[j, pl.ds(hv * 16, 16)]
                    accn = accn + s4 * d4
                    accc = accc + (s4 * r4) * d4
                    acce = acce + r4 * d4
                hn = _hsum(accn)
                hc = _hsum(accc)
                he = _hsum(acce)
                vn = jnp.where(lane == j16, hn, vn)
                vc = jnp.where(lane == j16, hc, vc)
                ve = jnp.where(lane == j16, he, ve)
                nsel = jnp.where(lane < 5, hn,
                                 jnp.where(lane < 10, hc, he))
                sub5 = jnp.where(lane < 5, 0,
                                 jnp.where(lane < 10, 5, 10))
                k = lane - sub5
                shift = jnp.where(k == 0, 0.0,
                                  jnp.where(k == 1, 600.0,
                                            jnp.where(k == 2, 1200.0,
                                                      jnp.where(k == 3,
                                                                1800.0,
                                                                2400.0))))
                x = (nsel - shift) * 0.25
                trow[j, pl.ds(0, 16)] = jnp.exp(jnp.minimum(x, 74.0))
                return vn, vc, ve

            zero = jnp.zeros((16,), jnp.float32)
            vn, vc, ve = lax.fori_loop(0, 16, edge, (zero, zero, zero))
            nbuf[0, pl.ds(g * 16, 16)] = vn
            nbuf[1, pl.ds(g * 16, 16)] = vc
            nbuf[2, pl.ds(g * 16, 16)] = ve
            return 0

        lax.fori_loop(0, C // 16, group, 0)
        pltpu.sync_copy(nbuf.at[0], n0_h.at[r])
        pltpu.sync_copy(nbuf.at[1], n1_h.at[r])
        pltpu.sync_copy(nbuf.at[2], n2_h.at[r])
        dd = pltpu.async_copy(trow, tacc.at[idx_ds], sem3, add=True)
        dd.wait()
        return 0

    lax.fori_loop(0, NCH, chunk, 0)
    plsc.subcore_barrier()
    pltpu.sync_copy(tacc.at[pl.ds(sid * RPT, RPT)],
                    tacc_h.at[cid, sid])


def _run_k1(ent, rel, src2, dstg2, dsts2, rid2):
    mesh = plsc.VectorSubcoreMesh(core_axis_name="c", subcore_axis_name="s")
    out_type = (
        jax.ShapeDtypeStruct((NROW, C), jnp.float32),
        jax.ShapeDtypeStruct((NROW, C), jnp.float32),
        jax.ShapeDtypeStruct((NROW, C), jnp.float32),
        jax.ShapeDtypeStruct((NC, NS, RPT, 16), jnp.float32),
    )
    scratch = [
        pltpu.VMEM((C,), jnp.int32),
        pltpu.VMEM((C,), jnp.int32),
        pltpu.VMEM((C,), jnp.int32),
        pltpu.VMEM((C,), jnp.int32),
        pltpu.VMEM((C, H), jnp.float32),
        pltpu.VMEM((C, H), jnp.float32),
        pltpu.VMEM((C, H), jnp.float32),
        pltpu.VMEM((3, C), jnp.float32),
        pltpu.VMEM((C, 16), jnp.float32),
        pltpu.VMEM((RPT, 16), jnp.float32),
        pltpu.VMEM_SHARED((ACC_R, 16), jnp.float32),
        pltpu.SemaphoreType.DMA,
        pltpu.SemaphoreType.DMA,
        pltpu.SemaphoreType.DMA,
        pltpu.SemaphoreType.DMA,
    ]
    f = pl.kernel(_k1_body, out_type=out_type, mesh=mesh,
                  scratch_types=scratch,
                  compiler_params=pltpu.CompilerParams(
                      use_tc_tiling_on_sc=False))
    return f(ent, rel, src2, dstg2, dsts2, rid2)


def _k2_body(t_ref, o_ref):
    t = t_ref[0] + t_ref[1]
    cols = []
    for sub in range(3):
        cands = []
        for k in range(5):
            tk = t[:, sub * 5 + k]
            cands.append(jnp.where(tk > 0.0,
                                   600.0 * k + 4.0 * jnp.log(tk), -3.0e38))
        mh = cands[0]
        for c in cands[1:]:
            mh = jnp.maximum(mh, c)
        cols.append(jnp.where(mh < -1.0e38, 0.0, mh))
    out = jnp.stack(cols, axis=-1)
    o_ref[...] = jnp.concatenate(
        [out, jnp.zeros((out.shape[0], 13), jnp.float32)], axis=-1)


def _run_k2(tacc):
    return pl.pallas_call(
        _k2_body,
        out_shape=jax.ShapeDtypeStruct((ACC_R, 16), jnp.float32),
    )(tacc)


def _k4_body(*refs):
    ent_qs = refs[0:NQ]
    rel_qs = refs[NQ:2 * NQ]
    (src_h, dsts_h, rid_h, n0_h, n1_h, n2_h, mhat_h,
     sacc_h, u_h, ebuf_h,
     idx_s, idx_ds, idx_r, nbuf, mrow, erow,
     rows_s, rows_r, stg_n, stg_c, stg_e, zbuf16, zbuf64,
     acc_n, acc_c, acc_e, sacc,
     sem0, sem1, sem2, sem3, sem4, sem5, sem6) = refs[2 * NQ:]
    cid = lax.axis_index("c")
    sid = lax.axis_index("s")
    wid = cid * NS + sid
    lane = _lanes()

    def zb(i, _):
        zbuf16[i, pl.ds(0, 16)] = jnp.zeros((16,), jnp.float32)
        return 0
    lax.fori_loop(0, RPT, zb, 0)

    def zb2(i, _):
        for hv in range(QH // 16):
            zbuf64[i, pl.ds(hv * 16, 16)] = jnp.zeros((16,), jnp.float32)
        return 0
    lax.fori_loop(0, RPT, zb2, 0)

    pltpu.sync_copy(zbuf16, sacc.at[pl.ds(sid * RPT, RPT)])

    for q in range(NQ):
        base0 = sid * RPT
        pltpu.sync_copy(zbuf64, acc_n.at[pl.ds(base0, RPT)])
        pltpu.sync_copy(zbuf64, acc_c.at[pl.ds(base0, RPT)])
        pltpu.sync_copy(zbuf64, acc_e.at[pl.ds(base0, RPT)])
        plsc.subcore_barrier()

        def chunk(ch, _):
            r = wid * NCH + ch
            pltpu.sync_copy(src_h.at[r], idx_s)
            pltpu.sync_copy(dsts_h.at[r], idx_ds)
            pltpu.sync_copy(rid_h.at[r], idx_r)
            ca = pltpu.async_copy(ent_qs[q].at[idx_s], rows_s, sem0)
            cb = pltpu.async_copy(rel_qs[q].at[idx_r], rows_r, sem1)
            if q == 0:
                pltpu.sync_copy(n0_h.at[r], nbuf.at[0])
                pltpu.sync_copy(n1_h.at[r], nbuf.at[1])
                pltpu.sync_copy(n2_h.at[r], nbuf.at[2])
                cm = pltpu.async_copy(mhat_h.at[idx_ds], mrow, sem2)
                cm.wait()
            else:
                ce = pltpu.async_copy(ebuf_h.at[r], erow, sem2)
                ce.wait()
            ca.wait()
            cb.wait()

            if q == 0:
                def egroup(g, _):
                    n0v = nbuf[0, pl.ds(g * 16, 16)]
                    n1v = nbuf[1, pl.ds(g * 16, 16)]
                    n2v = nbuf[2, pl.ds(g * 16, 16)]
                    for j16 in range(16):
                        j = g * 16 + j16
                        nv = jnp.where(lane == 0, n0v[j16],
                                       jnp.where(lane == 1, n1v[j16],
                                                 jnp.where(lane == 2,
                                                           n2v[j16], NEGB)))
                        mj = mrow[j, pl.ds(0, 16)]
                        erow[j, pl.ds(0, 16)] = jnp.exp(nv - mj)
                    return 0
                lax.fori_loop(0, C // 16, egroup, 0)
                ds1 = pltpu.async_copy(erow, sacc.at[idx_ds], sem3, add=True)
                ds2 = pltpu.async_copy(erow, ebuf_h.at[r], sem2)
                ds1.wait()
                ds2.wait()

            def mgroup(g, _):
                for j16 in range(16):
                    j = g * 16 + j16
                    ev = erow[j, pl.ds(0, 16)]
                    en = ev[0]
                    ec = ev[1]
                    ee = ev[2]
                    for hv in range(QH // 16):
                        s4 = rows_s[j, pl.ds(hv * 16, 16)]
                        r4 = rows_r[j, pl.ds(hv * 16, 16)]
                        stg_n[j, pl.ds(hv * 16, 16)] = s4 * en
                        stg_c[j, pl.ds(hv * 16, 16)] = (s4 * r4) * ec
                        stg_e[j, pl.ds(hv * 16, 16)] = r4 * ee
                return 0
            lax.fori_loop(0, C // 16, mgroup, 0)

            da = pltpu.async_copy(stg_n, acc_n.at[idx_ds], sem4, add=True)
            db = pltpu.async_copy(stg_c, acc_c.at[idx_ds], sem5, add=True)
            dc = pltpu.async_copy(stg_e, acc_e.at[idx_ds], sem6, add=True)
            da.wait()
            db.wait()
            dc.wait()
            return 0

        lax.fori_loop(0, NCH, chunk, 0)
        plsc.subcore_barrier()
        base = sid * RPT
        pltpu.sync_copy(acc_n.at[pl.ds(base, RPT)], u_h.at[q, cid, 0, sid])
        pltpu.sync_copy(acc_c.at[pl.ds(base, RPT)], u_h.at[q, cid, 1, sid])
        pltpu.sync_copy(acc_e.at[pl.ds(base, RPT)], u_h.at[q, cid, 2, sid])
        plsc.subcore_barrier()

    pltpu.sync_copy(sacc.at[pl.ds(sid * RPT, RPT)],
                    sacc_h.at[cid, sid])


def _run_k4(ent_qs, rel_qs, src2, dsts2, rid2, n0, n1, n2, mhat):
    mesh = plsc.VectorSubcoreMesh(core_axis_name="c", subcore_axis_name="s")
    out_type = (
        jax.ShapeDtypeStruct((NC, NS, RPT, 16), jnp.float32),
        jax.ShapeDtypeStruct((NQ, NC, 3, NS, RPT, QH), jnp.float32),
        jax.ShapeDtypeStruct((NROW, C, 16), jnp.float32),
    )
    scratch = [
        pltpu.VMEM((C,), jnp.int32),
        pltpu.VMEM((C,), jnp.int32),
        pltpu.VMEM((C,), jnp.int32),
        pltpu.VMEM((3, C), jnp.float32),
        pltpu.VMEM((C, 16), jnp.float32),
        pltpu.VMEM((C, 16), jnp.float32),
        pltpu.VMEM((C, QH), jnp.float32),
        pltpu.VMEM((C, QH), jnp.float32),
        pltpu.VMEM((C, QH), jnp.float32),
        pltpu.VMEM((C, QH), jnp.float32),
        pltpu.VMEM((C, QH), jnp.float32),
        pltpu.VMEM((RPT, 16), jnp.float32),
        pltpu.VMEM((RPT, QH), jnp.float32),
        pltpu.VMEM_SHARED((ACC_R, QH), jnp.float32),
        pltpu.VMEM_SHARED((ACC_R, QH), jnp.float32),
        pltpu.VMEM_SHARED((ACC_R, QH), jnp.float32),
        pltpu.VMEM_SHARED((ACC_R, 16), jnp.float32),
        pltpu.SemaphoreType.DMA,
        pltpu.SemaphoreType.DMA,
        pltpu.SemaphoreType.DMA,
        pltpu.SemaphoreType.DMA,
        pltpu.SemaphoreType.DMA,
        pltpu.SemaphoreType.DMA,
        pltpu.SemaphoreType.DMA,
    ]
    f = pl.kernel(_k4_body, out_type=out_type, mesh=mesh,
                  scratch_types=scratch,
                  compiler_params=pltpu.CompilerParams(
                      use_tc_tiling_on_sc=False))
    return f(*ent_qs, *rel_qs, src2, dsts2, rid2, n0, n1, n2, mhat)


def _k5_body(u_ref, s_ref, ent_ref, wn_ref, wc_ref, we_ref, o_ref):
    sp = s_ref[0] + s_ref[1]
    acc = ent_ref[...]
    for sub, w_ref in ((0, wn_ref), (1, wc_ref), (2, we_ref)):
        u = u_ref[:, 0, sub] + u_ref[:, 1, sub]
        neigh = jnp.concatenate([u[q] for q in range(NQ)], axis=-1)
        neigh = neigh / (sp[:, sub][:, None] + 1e-16)
        z = jnp.dot(neigh, w_ref[...], preferred_element_type=jnp.float32)
        acc = acc + jnp.tanh(z)
    o_ref[...] = acc


def _run_k5(u, sacc, ent, wn, wc, we):
    grid = APAD // BN
    return pl.pallas_call(
        _k5_body,
        grid=(grid,),
        in_specs=[
            pl.BlockSpec((NQ, NC, 3, BN, QH), lambda i: (0, 0, 0, i, 0)),
            pl.BlockSpec((NC, BN, 16), lambda i: (0, i, 0)),
            pl.BlockSpec((BN, H), lambda i: (i, 0)),
            pl.BlockSpec((H, H), lambda i: (0, 0)),
            pl.BlockSpec((H, H), lambda i: (0, 0)),
            pl.BlockSpec((H, H), lambda i: (0, 0)),
        ],
        out_specs=pl.BlockSpec((BN, H), lambda i: (i, 0)),
        out_shape=jax.ShapeDtypeStruct((APAD, H), jnp.float32),
    )(u, sacc, ent, wn, wc, we)


def kernel(edge_index, rel_id, ent_emb, rel_emb_0, rel_emb_1,
           W_edge_0, W_node_0, W_comp_0, W_edge_1, W_node_1, W_comp_1):
    src = edge_index[0]
    dst = edge_index[1]
    pad = EP - E
    srcp = jnp.concatenate([src, jnp.zeros((pad,), jnp.int32)])
    ridp = jnp.concatenate([rel_id, jnp.zeros((pad,), jnp.int32)])
    dstg = jnp.concatenate([dst, jnp.zeros((pad,), jnp.int32)])
    dsts = jnp.concatenate([dst, jnp.full((pad,), N, jnp.int32)])
    order = jnp.argsort(dsts)
    deal = lambda a: a[order].reshape(C, NROW).T.copy()
    src2 = deal(srcp)
    rid2 = deal(ridp)
    dstg2 = deal(dstg)
    dsts2 = deal(dsts)

    ent = jnp.pad(ent_emb, ((0, APAD - N), (0, 0)))
    layers = (
        (rel_emb_0, W_edge_0, W_node_0, W_comp_0),
        (rel_emb_1, W_edge_1, W_node_1, W_comp_1),
    )
    for rel, We, Wn, Wc in layers:
        n0, n1, n2, tacc = _run_k1(ent, rel, src2, dstg2, dsts2, rid2)
        mhat = _run_k2(tacc.reshape(NC, ACC_R, 16))
        ent_qs = tuple(ent[:, q * QH:(q + 1) * QH] for q in range(NQ))
        rel_qs = tuple(rel[:, q * QH:(q + 1) * QH] for q in range(NQ))
        sacc, u, _ = _run_k4(ent_qs, rel_qs, src2, dsts2, rid2,
                             n0, n1, n2, mhat)
        ent = _run_k5(u.reshape(NQ, NC, 3, ACC_R, QH),
                      sacc.reshape(NC, ACC_R, 16), ent, Wn, Wc, We)
    return ent[:N]


# speedup vs baseline: 3.4633x; 1.4308x over previous
import jax
import jax.numpy as jnp
from jax import lax
from jax.experimental import pallas as pl
from jax.experimental.pallas import tpu as pltpu
from jax.experimental.pallas import tpu_sc as plsc

N = 10000
NREL = 474
E = 160000
H = 256

NC = 2
NS = 16
NW = NC * NS
C = 112
NCH = 45
EPW = C * NCH
EP = EPW * NW
NROW = EP // C
ACC_R = 10240
RPT = ACC_R // NS
QH = 32
NQ = H // QH
APAD = 10240
BN = 512
NEGB = -1000.0


def _lanes():
    return lax.iota(jnp.int32, 16)


def _lg(x, idx):
    dnums = lax.GatherDimensionNumbers(
        offset_dims=(), collapsed_slice_dims=(0,), start_index_map=(0,))
    return lax.gather(x, idx[:, None], dnums, (1,),
                      mode=lax.GatherScatterMode.PROMISE_IN_BOUNDS)


def _hsum(x):
    lane = _lanes()
    for d in (1, 2, 4, 8):
        x = x + _lg(x, lane ^ d)
    return x


def _k1_body(ent_h, rel_h, src_h, dstg_h, dsts_h, rid_h,
             n0_h, n1_h, n2_h, tacc_h,
             idx_s, idx_dg, idx_ds, idx_r, rows_s, rows_d, rows_r,
             nbuf, trow, zbuf16, tacc, sem0, sem1, sem2, sem3):
    cid = lax.axis_index("c")
    sid = lax.axis_index("s")
    wid = cid * NS + sid
    lane = _lanes()

    def zb(i, _):
        zbuf16[i, pl.ds(0, 16)] = jnp.zeros((16,), jnp.float32)
        return 0
    lax.fori_loop(0, RPT // 8, zb, 0)

    def zc(k, _):
        pltpu.sync_copy(zbuf16, tacc.at[pl.ds(sid * RPT + k * (RPT // 8),
                                              RPT // 8)])
        return 0
    lax.fori_loop(0, 8, zc, 0)
    plsc.subcore_barrier()

    def chunk(ch, _):
        r = wid * NCH + ch
        pltpu.sync_copy(src_h.at[r], idx_s)
        pltpu.sync_copy(dstg_h.at[r], idx_dg)
        pltpu.sync_copy(dsts_h.at[r], idx_ds)
        pltpu.sync_copy(rid_h.at[r], idx_r)
        ca = pltpu.async_copy(ent_h.at[idx_s], rows_s, sem0)
        cb = pltpu.async_copy(ent_h.at[idx_dg], rows_d, sem1)
        cc = pltpu.async_copy(rel_h.at[idx_r], rows_r, sem2)
        ca.wait()
        cb.wait()
        cc.wait()

        def group(g, _):
            def edge(j16, carry):
                vn, vc, ve = carry
                j = g * 16 + j16
                accn = jnp.zeros((16,), jnp.float32)
                accc = jnp.zeros((16,), jnp.float32)
                acce = jnp.zeros((16,), jnp.float32)
                for hv in range(16):
                    s4 = rows_s[j, pl.ds(hv * 16, 16)]
                    d4 = rows_d[j, pl.ds(hv * 16, 16)]
                    r4 = rows_r[j, pl.ds(hv * 16, 16)]
                    accn = accn + s4 * d4
                    accc = accc + (s4 * r4) * d4
                    acce = acce + r4 * d4
                hn = _hsum(accn)
                hc = _hsum(accc)
                he = _hsum(acce)
                vn = jnp.where(lane == j16, hn, vn)
                vc = jnp.where(lane == j16, hc, vc)
                ve = jnp.where(lane == j16, he, ve)
                nsel = jnp.where(lane < 5, hn,
                                 jnp.where(lane < 10, hc, he))
                sub5 = jnp.where(lane < 5, 0,
                                 jnp.where(lane < 10, 5, 10))
                k = lane - sub5
                shift = jnp.where(k == 0, 0.0,
                                  jnp.where(k == 1, 600.0,
                                            jnp.where(k == 2, 1200.0,
                                                      jnp.where(k == 3,
                                                                1800.0,
                                                                2400.0))))
                x = (nsel - shift) * 0.25
                trow[j, pl.ds(0, 16)] = jnp.exp(jnp.minimum(x, 74.0))
                return vn, vc, ve

            zero = jnp.zeros((16,), jnp.float32)
            vn, vc, ve = lax.fori_loop(0, 16, edge, (zero, zero, zero))
            nbuf[0, pl.ds(g * 16, 16)] = vn
            nbuf[1, pl.ds(g * 16, 16)] = vc
            nbuf[2, pl.ds(g * 16, 16)] = ve
            return 0

        lax.fori_loop(0, C // 16, group, 0)
        pltpu.sync_copy(nbuf.at[0], n0_h.at[r])
        pltpu.sync_copy(nbuf.at[1], n1_h.at[r])
        pltpu.sync_copy(nbuf.at[2], n2_h.at[r])
        dd = pltpu.async_copy(trow, tacc.at[idx_ds], sem3, add=True)
        dd.wait()
        return 0

    lax.fori_loop(0, NCH, chunk, 0)
    plsc.subcore_barrier()
    pltpu.sync_copy(tacc.at[pl.ds(sid * RPT, RPT)],
                    tacc_h.at[cid, sid])


def _run_k1(ent, rel, src2, dstg2, dsts2, rid2):
    mesh = plsc.VectorSubcoreMesh(core_axis_name="c", subcore_axis_name="s")
    out_type = (
        jax.ShapeDtypeStruct((NROW, C), jnp.float32),
        jax.ShapeDtypeStruct((NROW, C), jnp.float32),
        jax.ShapeDtypeStruct((NROW, C), jnp.float32),
        jax.ShapeDtypeStruct((NC, NS, RPT, 16), jnp.float32),
    )
    scratch = [
        pltpu.VMEM((C,), jnp.int32),
        pltpu.VMEM((C,), jnp.int32),
        pltpu.VMEM((C,), jnp.int32),
        pltpu.VMEM((C,), jnp.int32),
        pltpu.VMEM((C, H), jnp.float32),
        pltpu.VMEM((C, H), jnp.float32),
        pltpu.VMEM((C, H), jnp.float32),
        pltpu.VMEM((3, C), jnp.float32),
        pltpu.VMEM((C, 16), jnp.float32),
        pltpu.VMEM((RPT // 8, 16), jnp.float32),
        pltpu.VMEM_SHARED((ACC_R, 16), jnp.float32),
        pltpu.SemaphoreType.DMA,
        pltpu.SemaphoreType.DMA,
        pltpu.SemaphoreType.DMA,
        pltpu.SemaphoreType.DMA,
    ]
    f = pl.kernel(_k1_body, out_type=out_type, mesh=mesh,
                  scratch_types=scratch,
                  compiler_params=pltpu.CompilerParams(
                      use_tc_tiling_on_sc=False))
    return f(ent, rel, src2, dstg2, dsts2, rid2)


def _k2_body(t_ref, o_ref):
    t = t_ref[0] + t_ref[1]
    cols = []
    for sub in range(3):
        cands = []
        for k in range(5):
            tk = t[:, sub * 5 + k]
            cands.append(jnp.where(tk > 0.0,
                                   600.0 * k + 4.0 * jnp.log(tk), -3.0e38))
        mh = cands[0]
        for c in cands[1:]:
            mh = jnp.maximum(mh, c)
        cols.append(jnp.where(mh < -1.0e38, 0.0, mh))
    out = jnp.stack(cols, axis=-1)
    o_ref[...] = jnp.concatenate(
        [out, jnp.zeros((out.shape[0], 13), jnp.float32)], axis=-1)


def _run_k2(tacc):
    return pl.pallas_call(
        _k2_body,
        out_shape=jax.ShapeDtypeStruct((ACC_R, 16), jnp.float32),
    )(tacc)


def _k4_body(*refs):
    ent_qs = refs[0:NQ]
    rel_qs = refs[NQ:2 * NQ]
    (src_h, dsts_h, rid_h, n0_h, n1_h, n2_h, mhat_h,
     sacc_h, u_h, ebuf_h,
     idx_s, idx_ds, idx_r, nbuf, mrow, erow,
     rows_s, rows_r, stg_n, stg_c, stg_e, zbuf16, zbuf64,
     acc_n, acc_c, acc_e, sacc,
     sem0, sem1, sem2, sem3, sem4, sem5, sem6) = refs[2 * NQ:]
    cid = lax.axis_index("c")
    sid = lax.axis_index("s")
    wid = cid * NS + sid
    lane = _lanes()

    def zb(i, _):
        zbuf16[i, pl.ds(0, 16)] = jnp.zeros((16,), jnp.float32)
        return 0
    lax.fori_loop(0, RPT // 8, zb, 0)

    def zb2(i, _):
        for hv in range(QH // 16):
            zbuf64[i, pl.ds(hv * 16, 16)] = jnp.zeros((16,), jnp.float32)
        return 0
    lax.fori_loop(0, RPT // 8, zb2, 0)

    def zs(k, _):
        pltpu.sync_copy(zbuf16, sacc.at[pl.ds(sid * RPT + k * (RPT // 8),
                                              RPT // 8)])
        return 0
    lax.fori_loop(0, 8, zs, 0)

    for q in range(NQ):
        def za(k, _):
            b = sid * RPT + k * (RPT // 8)
            pltpu.sync_copy(zbuf64, acc_n.at[pl.ds(b, RPT // 8)])
            pltpu.sync_copy(zbuf64, acc_c.at[pl.ds(b, RPT // 8)])
            pltpu.sync_copy(zbuf64, acc_e.at[pl.ds(b, RPT // 8)])
            return 0
        lax.fori_loop(0, 8, za, 0)
        plsc.subcore_barrier()

        def chunk(ch, _):
            r = wid * NCH + ch
            pltpu.sync_copy(src_h.at[r], idx_s)
            pltpu.sync_copy(dsts_h.at[r], idx_ds)
            pltpu.sync_copy(rid_h.at[r], idx_r)
            ca = pltpu.async_copy(ent_qs[q].at[idx_s], rows_s, sem0)
            cb = pltpu.async_copy(rel_qs[q].at[idx_r], rows_r, sem1)
            if q == 0:
                pltpu.sync_copy(n0_h.at[r], nbuf.at[0])
                pltpu.sync_copy(n1_h.at[r], nbuf.at[1])
                pltpu.sync_copy(n2_h.at[r], nbuf.at[2])
                cm = pltpu.async_copy(mhat_h.at[idx_ds], mrow, sem2)
                cm.wait()
            else:
                ce = pltpu.async_copy(ebuf_h.at[r], erow, sem2)
                ce.wait()
            ca.wait()
            cb.wait()

            if q == 0:
                def egroup(g, _):
                    n0v = nbuf[0, pl.ds(g * 16, 16)]
                    n1v = nbuf[1, pl.ds(g * 16, 16)]
                    n2v = nbuf[2, pl.ds(g * 16, 16)]
                    for j16 in range(16):
                        j = g * 16 + j16
                        nv = jnp.where(lane == 0, n0v[j16],
                                       jnp.where(lane == 1, n1v[j16],
                                                 jnp.where(lane == 2,
                                                           n2v[j16], NEGB)))
                        mj = mrow[j, pl.ds(0, 16)]
                        erow[j, pl.ds(0, 16)] = jnp.exp(nv - mj)
                    return 0
                lax.fori_loop(0, C // 16, egroup, 0)
                ds1 = pltpu.async_copy(erow, sacc.at[idx_ds], sem3, add=True)
                ds2 = pltpu.async_copy(erow, ebuf_h.at[r], sem2)
                ds1.wait()
                ds2.wait()

            def mgroup(g, _):
                for j16 in range(16):
                    j = g * 16 + j16
                    ev = erow[j, pl.ds(0, 16)]
                    en = ev[0]
                    ec = ev[1]
                    ee = ev[2]
                    for hv in range(QH // 16):
                        s4 = rows_s[j, pl.ds(hv * 16, 16)]
                        r4 = rows_r[j, pl.ds(hv * 16, 16)]
                        stg_n[j, pl.ds(hv * 16, 16)] = s4 * en
                        stg_c[j, pl.ds(hv * 16, 16)] = (s4 * r4) * ec
                        stg_e[j, pl.ds(hv * 16, 16)] = r4 * ee
                return 0
            lax.fori_loop(0, C // 16, mgroup, 0)

            da = pltpu.async_copy(stg_n, acc_n.at[idx_ds], sem4, add=True)
            db = pltpu.async_copy(stg_c, acc_c.at[idx_ds], sem5, add=True)
            dc = pltpu.async_copy(stg_e, acc_e.at[idx_ds], sem6, add=True)
            da.wait()
            db.wait()
            dc.wait()
            return 0

        lax.fori_loop(0, NCH, chunk, 0)
        plsc.subcore_barrier()
        base = sid * RPT
        pltpu.sync_copy(acc_n.at[pl.ds(base, RPT)], u_h.at[q, cid, 0, sid])
        pltpu.sync_copy(acc_c.at[pl.ds(base, RPT)], u_h.at[q, cid, 1, sid])
        pltpu.sync_copy(acc_e.at[pl.ds(base, RPT)], u_h.at[q, cid, 2, sid])
        plsc.subcore_barrier()

    pltpu.sync_copy(sacc.at[pl.ds(sid * RPT, RPT)],
                    sacc_h.at[cid, sid])


def _run_k4(ent_qs, rel_qs, src2, dsts2, rid2, n0, n1, n2, mhat):
    mesh = plsc.VectorSubcoreMesh(core_axis_name="c", subcore_axis_name="s")
    out_type = (
        jax.ShapeDtypeStruct((NC, NS, RPT, 16), jnp.float32),
        jax.ShapeDtypeStruct((NQ, NC, 3, NS, RPT, QH), jnp.float32),
        jax.ShapeDtypeStruct((NROW, C, 16), jnp.float32),
    )
    scratch = [
        pltpu.VMEM((C,), jnp.int32),
        pltpu.VMEM((C,), jnp.int32),
        pltpu.VMEM((C,), jnp.int32),
        pltpu.VMEM((3, C), jnp.float32),
        pltpu.VMEM((C, 16), jnp.float32),
        pltpu.VMEM((C, 16), jnp.float32),
        pltpu.VMEM((C, QH), jnp.float32),
        pltpu.VMEM((C, QH), jnp.float32),
        pltpu.VMEM((C, QH), jnp.float32),
        pltpu.VMEM((C, QH), jnp.float32),
        pltpu.VMEM((C, QH), jnp.float32),
        pltpu.VMEM((RPT // 8, 16), jnp.float32),
        pltpu.VMEM((RPT // 8, QH), jnp.float32),
        pltpu.VMEM_SHARED((ACC_R, QH), jnp.float32),
        pltpu.VMEM_SHARED((ACC_R, QH), jnp.float32),
        pltpu.VMEM_SHARED((ACC_R, QH), jnp.float32),
        pltpu.VMEM_SHARED((ACC_R, 16), jnp.float32),
        pltpu.SemaphoreType.DMA,
        pltpu.SemaphoreType.DMA,
        pltpu.SemaphoreType.DMA,
        pltpu.SemaphoreType.DMA,
        pltpu.SemaphoreType.DMA,
        pltpu.SemaphoreType.DMA,
        pltpu.SemaphoreType.DMA,
    ]
    f = pl.kernel(_k4_body, out_type=out_type, mesh=mesh,
                  scratch_types=scratch,
                  compiler_params=pltpu.CompilerParams(
                      use_tc_tiling_on_sc=False))
    return f(*ent_qs, *rel_qs, src2, dsts2, rid2, n0, n1, n2, mhat)


def _k5_body(u_ref, s_ref, ent_ref, wn_ref, wc_ref, we_ref, o_ref):
    sp = s_ref[0] + s_ref[1]
    acc = ent_ref[...]
    for sub, w_ref in ((0, wn_ref), (1, wc_ref), (2, we_ref)):
        u = u_ref[:, 0, sub] + u_ref[:, 1, sub]
        neigh = jnp.concatenate([u[q] for q in range(NQ)], axis=-1)
        neigh = neigh / (sp[:, sub][:, None] + 1e-16)
        z = jnp.dot(neigh, w_ref[...], preferred_element_type=jnp.float32)
        acc = acc + jnp.tanh(z)
    o_ref[...] = acc


def _run_k5(u, sacc, ent, wn, wc, we):
    grid = APAD // BN
    return pl.pallas_call(
        _k5_body,
        grid=(grid,),
        in_specs=[
            pl.BlockSpec((NQ, NC, 3, BN, QH), lambda i: (0, 0, 0, i, 0)),
            pl.BlockSpec((NC, BN, 16), lambda i: (0, i, 0)),
            pl.BlockSpec((BN, H), lambda i: (i, 0)),
            pl.BlockSpec((H, H), lambda i: (0, 0)),
            pl.BlockSpec((H, H), lambda i: (0, 0)),
            pl.BlockSpec((H, H), lambda i: (0, 0)),
        ],
        out_specs=pl.BlockSpec((BN, H), lambda i: (i, 0)),
        out_shape=jax.ShapeDtypeStruct((APAD, H), jnp.float32),
    )(u, sacc, ent, wn, wc, we)


def kernel(edge_index, rel_id, ent_emb, rel_emb_0, rel_emb_1,
           W_edge_0, W_node_0, W_comp_0, W_edge_1, W_node_1, W_comp_1):
    src = edge_index[0]
    dst = edge_index[1]
    pad = EP - E
    srcp = jnp.concatenate([src, jnp.zeros((pad,), jnp.int32)])
    ridp = jnp.concatenate([rel_id, jnp.zeros((pad,), jnp.int32)])
    dstg = jnp.concatenate([dst, jnp.zeros((pad,), jnp.int32)])
    dsts = jnp.concatenate([dst, jnp.full((pad,), N, jnp.int32)])
    order = jnp.argsort(dsts)
    deal = lambda a: a[order].reshape(C, NROW).T.copy()
    src2 = deal(srcp)
    rid2 = deal(ridp)
    dstg2 = deal(dstg)
    dsts2 = deal(dsts)

    ent = jnp.pad(ent_emb, ((0, APAD - N), (0, 0)))
    layers = (
        (rel_emb_0, W_edge_0, W_node_0, W_comp_0),
        (rel_emb_1, W_edge_1, W_node_1, W_comp_1),
    )
    for rel, We, Wn, Wc in layers:
        n0, n1, n2, tacc = _run_k1(ent, rel, src2, dstg2, dsts2, rid2)
        mhat = _run_k2(tacc.reshape(NC, ACC_R, 16))
        ent_qs = tuple(ent[:, q * QH:(q + 1) * QH] for q in range(NQ))
        rel_qs = tuple(rel[:, q * QH:(q + 1) * QH] for q in range(NQ))
        sacc, u, _ = _run_k4(ent_qs, rel_qs, src2, dsts2, rid2,
                             n0, n1, n2, mhat)
        ent = _run_k5(u.reshape(NQ, NC, 3, ACC_R, QH),
                      sacc.reshape(NC, ACC_R, 16), ent, Wn, Wc, We)
    return ent[:N]


# speedup vs baseline: 4.4206x; 1.2764x over previous
import jax
import jax.numpy as jnp
from jax import lax
from jax.experimental import pallas as pl
from jax.experimental.pallas import tpu as pltpu
from jax.experimental.pallas import tpu_sc as plsc

N = 10000
NREL = 474
E = 160000
H = 256

NC = 2
NS = 16
NW = NC * NS
C = 112
NCH = 45
EPW = C * NCH
EP = EPW * NW
NROW = EP // C
ACC_R = 10240
RPT = ACC_R // NS
QH = 32
NQ = H // QH
APAD = 10240
BN = 512
NEGB = -1000.0


def _lanes():
    return lax.iota(jnp.int32, 16)


def _lg(x, idx):
    dnums = lax.GatherDimensionNumbers(
        offset_dims=(), collapsed_slice_dims=(0,), start_index_map=(0,))
    return lax.gather(x, idx[:, None], dnums, (1,),
                      mode=lax.GatherScatterMode.PROMISE_IN_BOUNDS)


def _hsum(x):
    lane = _lanes()
    for d in (1, 2, 4, 8):
        x = x + _lg(x, lane ^ d)
    return x


def _k1_body(ent_h, rel_h, src_h, dstg_h, dsts_h, rid_h,
             n0_h, n1_h, n2_h, tacc_h,
             idx_s, idx_dg, idx_ds, idx_r, rows_s, rows_d, rows_r,
             nbuf, trow, zbuf16, tacc, sem0, sem1, sem2, sem3):
    cid = lax.axis_index("c")
    sid = lax.axis_index("s")
    wid = cid * NS + sid
    lane = _lanes()

    def zb(i, _):
        zbuf16[i, pl.ds(0, 16)] = jnp.zeros((16,), jnp.float32)
        return 0
    lax.fori_loop(0, RPT // 8, zb, 0)

    def zc(k, _):
        pltpu.sync_copy(zbuf16, tacc.at[pl.ds(sid * RPT + k * (RPT // 8),
                                              RPT // 8)])
        return 0
    lax.fori_loop(0, 8, zc, 0)
    plsc.subcore_barrier()

    pltpu.sync_copy(src_h.at[wid], idx_s)
    pltpu.sync_copy(dstg_h.at[wid], idx_dg)
    pltpu.sync_copy(dsts_h.at[wid], idx_ds)
    pltpu.sync_copy(rid_h.at[wid], idx_r)

    def chunk(ch, _):
        r = wid * NCH + ch
        ca = pltpu.async_copy(ent_h.at[idx_s.at[pl.ds(ch * C, C)]],
                              rows_s, sem0)
        cb = pltpu.async_copy(ent_h.at[idx_dg.at[pl.ds(ch * C, C)]],
                              rows_d, sem1)
        cc = pltpu.async_copy(rel_h.at[idx_r.at[pl.ds(ch * C, C)]],
                              rows_r, sem2)
        ca.wait()
        cb.wait()
        cc.wait()

        def group(g, _):
            def edge(j16, carry):
                vn, vc, ve = carry
                j = g * 16 + j16
                accn = jnp.zeros((16,), jnp.float32)
                accc = jnp.zeros((16,), jnp.float32)
                acce = jnp.zeros((16,), jnp.float32)
                for hv in range(16):
                    s4 = rows_s[j, pl.ds(hv * 16, 16)]
                    d4 = rows_d[j, pl.ds(hv * 16, 16)]
                    r4 = rows_r[j, pl.ds(hv * 16, 16)]
                    accn = accn + s4 * d4
                    accc = accc + (s4 * r4) * d4
                    acce = acce + r4 * d4
                hn = _hsum(accn)
                hc = _hsum(accc)
                he = _hsum(acce)
                vn = jnp.where(lane == j16, hn, vn)
                vc = jnp.where(lane == j16, hc, vc)
                ve = jnp.where(lane == j16, he, ve)
                nsel = jnp.where(lane < 5, hn,
                                 jnp.where(lane < 10, hc, he))
                sub5 = jnp.where(lane < 5, 0,
                                 jnp.where(lane < 10, 5, 10))
                k = lane - sub5
                shift = jnp.where(k == 0, 0.0,
                                  jnp.where(k == 1, 600.0,
                                            jnp.where(k == 2, 1200.0,
                                                      jnp.where(k == 3,
                                                                1800.0,
                                                                2400.0))))
                x = (nsel - shift) * 0.25
                trow[j, pl.ds(0, 16)] = jnp.exp(jnp.minimum(x, 74.0))
                return vn, vc, ve

            zero = jnp.zeros((16,), jnp.float32)
            vn, vc, ve = lax.fori_loop(0, 16, edge, (zero, zero, zero))
            nbuf[0, pl.ds(g * 16, 16)] = vn
            nbuf[1, pl.ds(g * 16, 16)] = vc
            nbuf[2, pl.ds(g * 16, 16)] = ve
            return 0

        lax.fori_loop(0, C // 16, group, 0)
        pltpu.sync_copy(nbuf.at[0], n0_h.at[r])
        pltpu.sync_copy(nbuf.at[1], n1_h.at[r])
        pltpu.sync_copy(nbuf.at[2], n2_h.at[r])
        dd = pltpu.async_copy(trow, tacc.at[idx_ds.at[pl.ds(ch * C, C)]],
                              sem3, add=True)
        dd.wait()
        return 0

    lax.fori_loop(0, NCH, chunk, 0)
    plsc.subcore_barrier()
    pltpu.sync_copy(tacc.at[pl.ds(sid * RPT, RPT)],
                    tacc_h.at[cid, sid])


def _run_k1(ent, rel, src2, dstg2, dsts2, rid2):
    mesh = plsc.VectorSubcoreMesh(core_axis_name="c", subcore_axis_name="s")
    out_type = (
        jax.ShapeDtypeStruct((NROW, C), jnp.float32),
        jax.ShapeDtypeStruct((NROW, C), jnp.float32),
        jax.ShapeDtypeStruct((NROW, C), jnp.float32),
        jax.ShapeDtypeStruct((NC, NS, RPT, 16), jnp.float32),
    )
    scratch = [
        pltpu.VMEM((EPW,), jnp.int32),
        pltpu.VMEM((EPW,), jnp.int32),
        pltpu.VMEM((EPW,), jnp.int32),
        pltpu.VMEM((EPW,), jnp.int32),
        pltpu.VMEM((C, H), jnp.float32),
        pltpu.VMEM((C, H), jnp.float32),
        pltpu.VMEM((C, H), jnp.float32),
        pltpu.VMEM((3, C), jnp.float32),
        pltpu.VMEM((C, 16), jnp.float32),
        pltpu.VMEM((RPT // 8, 16), jnp.float32),
        pltpu.VMEM_SHARED((ACC_R, 16), jnp.float32),
        pltpu.SemaphoreType.DMA,
        pltpu.SemaphoreType.DMA,
        pltpu.SemaphoreType.DMA,
        pltpu.SemaphoreType.DMA,
    ]
    f = pl.kernel(_k1_body, out_type=out_type, mesh=mesh,
                  scratch_types=scratch,
                  compiler_params=pltpu.CompilerParams(
                      use_tc_tiling_on_sc=False))
    return f(ent, rel, src2, dstg2, dsts2, rid2)


def _k2_body(t_ref, o_ref):
    t = t_ref[0] + t_ref[1]
    cols = []
    for sub in range(3):
        cands = []
        for k in range(5):
            tk = t[:, sub * 5 + k]
            cands.append(jnp.where(tk > 0.0,
                                   600.0 * k + 4.0 * jnp.log(tk), -3.0e38))
        mh = cands[0]
        for c in cands[1:]:
            mh = jnp.maximum(mh, c)
        cols.append(jnp.where(mh < -1.0e38, 0.0, mh))
    out = jnp.stack(cols, axis=-1)
    o_ref[...] = jnp.concatenate(
        [out, jnp.zeros((out.shape[0], 13), jnp.float32)], axis=-1)


def _run_k2(tacc):
    return pl.pallas_call(
        _k2_body,
        out_shape=jax.ShapeDtypeStruct((ACC_R, 16), jnp.float32),
    )(tacc)


def _k4_body(*refs):
    ent_qs = refs[0:NQ]
    rel_qs = refs[NQ:2 * NQ]
    (src_h, dsts_h, rid_h, n0_h, n1_h, n2_h, mhat_h,
     sacc_h, u_h, ebuf_h,
     idx_s, idx_ds, idx_r, nbuf, mrow, erow,
     rows_s, rows_r, stg_n, stg_c, stg_e, zbuf16, zbuf64,
     acc_n, acc_c, acc_e, sacc,
     sem0, sem1, sem2, sem3, sem4, sem5, sem6) = refs[2 * NQ:]
    cid = lax.axis_index("c")
    sid = lax.axis_index("s")
    wid = cid * NS + sid
    lane = _lanes()
    pltpu.sync_copy(src_h.at[wid], idx_s)
    pltpu.sync_copy(dsts_h.at[wid], idx_ds)
    pltpu.sync_copy(rid_h.at[wid], idx_r)

    def zb(i, _):
        zbuf16[i, pl.ds(0, 16)] = jnp.zeros((16,), jnp.float32)
        return 0
    lax.fori_loop(0, RPT // 8, zb, 0)

    def zb2(i, _):
        for hv in range(QH // 16):
            zbuf64[i, pl.ds(hv * 16, 16)] = jnp.zeros((16,), jnp.float32)
        return 0
    lax.fori_loop(0, RPT // 8, zb2, 0)

    def zs(k, _):
        pltpu.sync_copy(zbuf16, sacc.at[pl.ds(sid * RPT + k * (RPT // 8),
                                              RPT // 8)])
        return 0
    lax.fori_loop(0, 8, zs, 0)

    for q in range(NQ):
        def za(k, _):
            b = sid * RPT + k * (RPT // 8)
            pltpu.sync_copy(zbuf64, acc_n.at[pl.ds(b, RPT // 8)])
            pltpu.sync_copy(zbuf64, acc_c.at[pl.ds(b, RPT // 8)])
            pltpu.sync_copy(zbuf64, acc_e.at[pl.ds(b, RPT // 8)])
            return 0
        lax.fori_loop(0, 8, za, 0)
        plsc.subcore_barrier()

        def chunk(ch, _):
            r = wid * NCH + ch
            ca = pltpu.async_copy(ent_qs[q].at[idx_s.at[pl.ds(ch * C, C)]],
                                  rows_s, sem0)
            cb = pltpu.async_copy(rel_qs[q].at[idx_r.at[pl.ds(ch * C, C)]],
                                  rows_r, sem1)
            if q == 0:
                pltpu.sync_copy(n0_h.at[r], nbuf.at[0])
                pltpu.sync_copy(n1_h.at[r], nbuf.at[1])
                pltpu.sync_copy(n2_h.at[r], nbuf.at[2])
                cm = pltpu.async_copy(
                    mhat_h.at[idx_ds.at[pl.ds(ch * C, C)]], mrow, sem2)
                cm.wait()
            else:
                ce = pltpu.async_copy(ebuf_h.at[r], erow, sem2)
                ce.wait()
            ca.wait()
            cb.wait()

            if q == 0:
                def egroup(g, _):
                    n0v = nbuf[0, pl.ds(g * 16, 16)]
                    n1v = nbuf[1, pl.ds(g * 16, 16)]
                    n2v = nbuf[2, pl.ds(g * 16, 16)]
                    for j16 in range(16):
                        j = g * 16 + j16
                        nv = jnp.where(lane == 0, n0v[j16],
                                       jnp.where(lane == 1, n1v[j16],
                                                 jnp.where(lane == 2,
                                                           n2v[j16], NEGB)))
                        mj = mrow[j, pl.ds(0, 16)]
                        erow[j, pl.ds(0, 16)] = jnp.exp(nv - mj)
                    return 0
                lax.fori_loop(0, C // 16, egroup, 0)
                ds1 = pltpu.async_copy(
                    erow, sacc.at[idx_ds.at[pl.ds(ch * C, C)]],
                    sem3, add=True)
                ds2 = pltpu.async_copy(erow, ebuf_h.at[r], sem2)
                ds1.wait()
                ds2.wait()

            def mgroup(g, _):
                for j16 in range(16):
                    j = g * 16 + j16
                    ev = erow[j, pl.ds(0, 16)]
                    en = ev[0]
                    ec = ev[1]
                    ee = ev[2]
                    for hv in range(QH // 16):
                        s4 = rows_s[j, pl.ds(hv * 16, 16)]
                        r4 = rows_r[j, pl.ds(hv * 16, 16)]
                        stg_n[j, pl.ds(hv * 16, 16)] = s4 * en
                        stg_c[j, pl.ds(hv * 16, 16)] = (s4 * r4) * ec
                        stg_e[j, pl.ds(hv * 16, 16)] = r4 * ee
                return 0
            lax.fori_loop(0, C // 16, mgroup, 0)

            ids = idx_ds.at[pl.ds(ch * C, C)]
            da = pltpu.async_copy(stg_n, acc_n.at[ids], sem4, add=True)
            db = pltpu.async_copy(stg_c, acc_c.at[ids], sem5, add=True)
            dc = pltpu.async_copy(stg_e, acc_e.at[ids], sem6, add=True)
            da.wait()
            db.wait()
            dc.wait()
            return 0

        lax.fori_loop(0, NCH, chunk, 0)
        plsc.subcore_barrier()
        base = sid * RPT
        pltpu.sync_copy(acc_n.at[pl.ds(base, RPT)], u_h.at[q, cid, 0, sid])
        pltpu.sync_copy(acc_c.at[pl.ds(base, RPT)], u_h.at[q, cid, 1, sid])
        pltpu.sync_copy(acc_e.at[pl.ds(base, RPT)], u_h.at[q, cid, 2, sid])
        plsc.subcore_barrier()

    pltpu.sync_copy(sacc.at[pl.ds(sid * RPT, RPT)],
                    sacc_h.at[cid, sid])


def _run_k4(ent_qs, rel_qs, src2, dsts2, rid2, n0, n1, n2, mhat):
    mesh = plsc.VectorSubcoreMesh(core_axis_name="c", subcore_axis_name="s")
    out_type = (
        jax.ShapeDtypeStruct((NC, NS, RPT, 16), jnp.float32),
        jax.ShapeDtypeStruct((NQ, NC, 3, NS, RPT, QH), jnp.float32),
        jax.ShapeDtypeStruct((NROW, C, 16), jnp.float32),
    )
    scratch = [
        pltpu.VMEM((EPW,), jnp.int32),
        pltpu.VMEM((EPW,), jnp.int32),
        pltpu.VMEM((EPW,), jnp.int32),
        pltpu.VMEM((3, C), jnp.float32),
        pltpu.VMEM((C, 16), jnp.float32),
        pltpu.VMEM((C, 16), jnp.float32),
        pltpu.VMEM((C, QH), jnp.float32),
        pltpu.VMEM((C, QH), jnp.float32),
        pltpu.VMEM((C, QH), jnp.float32),
        pltpu.VMEM((C, QH), jnp.float32),
        pltpu.VMEM((C, QH), jnp.float32),
        pltpu.VMEM((RPT // 8, 16), jnp.float32),
        pltpu.VMEM((RPT // 8, QH), jnp.float32),
        pltpu.VMEM_SHARED((ACC_R, QH), jnp.float32),
        pltpu.VMEM_SHARED((ACC_R, QH), jnp.float32),
        pltpu.VMEM_SHARED((ACC_R, QH), jnp.float32),
        pltpu.VMEM_SHARED((ACC_R, 16), jnp.float32),
        pltpu.SemaphoreType.DMA,
        pltpu.SemaphoreType.DMA,
        pltpu.SemaphoreType.DMA,
        pltpu.SemaphoreType.DMA,
        pltpu.SemaphoreType.DMA,
        pltpu.SemaphoreType.DMA,
        pltpu.SemaphoreType.DMA,
    ]
    f = pl.kernel(_k4_body, out_type=out_type, mesh=mesh,
                  scratch_types=scratch,
                  compiler_params=pltpu.CompilerParams(
                      use_tc_tiling_on_sc=False))
    return f(*ent_qs, *rel_qs, src2, dsts2, rid2, n0, n1, n2, mhat)


def _k5_body(u_ref, s_ref, ent_ref, wn_ref, wc_ref, we_ref, o_ref):
    sp = s_ref[0] + s_ref[1]
    acc = ent_ref[...]
    for sub, w_ref in ((0, wn_ref), (1, wc_ref), (2, we_ref)):
        u = u_ref[:, 0, sub] + u_ref[:, 1, sub]
        neigh = jnp.concatenate([u[q] for q in range(NQ)], axis=-1)
        neigh = neigh / (sp[:, sub][:, None] + 1e-16)
        z = jnp.dot(neigh, w_ref[...], preferred_element_type=jnp.float32)
        acc = acc + jnp.tanh(z)
    o_ref[...] = acc


def _run_k5(u, sacc, ent, wn, wc, we):
    grid = APAD // BN
    return pl.pallas_call(
        _k5_body,
        grid=(grid,),
        in_specs=[
            pl.BlockSpec((NQ, NC, 3, BN, QH), lambda i: (0, 0, 0, i, 0)),
            pl.BlockSpec((NC, BN, 16), lambda i: (0, i, 0)),
            pl.BlockSpec((BN, H), lambda i: (i, 0)),
            pl.BlockSpec((H, H), lambda i: (0, 0)),
            pl.BlockSpec((H, H), lambda i: (0, 0)),
            pl.BlockSpec((H, H), lambda i: (0, 0)),
        ],
        out_specs=pl.BlockSpec((BN, H), lambda i: (i, 0)),
        out_shape=jax.ShapeDtypeStruct((APAD, H), jnp.float32),
    )(u, sacc, ent, wn, wc, we)


def kernel(edge_index, rel_id, ent_emb, rel_emb_0, rel_emb_1,
           W_edge_0, W_node_0, W_comp_0, W_edge_1, W_node_1, W_comp_1):
    src = edge_index[0]
    dst = edge_index[1]
    pad = EP - E
    srcp = jnp.concatenate([src, jnp.zeros((pad,), jnp.int32)])
    ridp = jnp.concatenate([rel_id, jnp.zeros((pad,), jnp.int32)])
    dstg = jnp.concatenate([dst, jnp.zeros((pad,), jnp.int32)])
    dsts = jnp.concatenate([dst, jnp.full((pad,), N, jnp.int32)])
    order = jnp.argsort(dsts)
    deal = lambda a: a[order].reshape(C, NROW).T.reshape(NW, EPW).copy()
    src2 = deal(srcp)
    rid2 = deal(ridp)
    dstg2 = deal(dstg)
    dsts2 = deal(dsts)

    ent = jnp.pad(ent_emb, ((0, APAD - N), (0, 0)))
    layers = (
        (rel_emb_0, W_edge_0, W_node_0, W_comp_0),
        (rel_emb_1, W_edge_1, W_node_1, W_comp_1),
    )
    for rel, We, Wn, Wc in layers:
        n0, n1, n2, tacc = _run_k1(ent, rel, src2, dstg2, dsts2, rid2)
        mhat = _run_k2(tacc.reshape(NC, ACC_R, 16))
        ent_qs = tuple(ent[:, q * QH:(q + 1) * QH] for q in range(NQ))
        rel_qs = tuple(rel[:, q * QH:(q + 1) * QH] for q in range(NQ))
        sacc, u, _ = _run_k4(ent_qs, rel_qs, src2, dsts2, rid2,
                             n0, n1, n2, mhat)
        ent = _run_k5(u.reshape(NQ, NC, 3, ACC_R, QH),
                      sacc.reshape(NC, ACC_R, 16), ent, Wn, Wc, We)
    return ent[:N]


# speedup vs baseline: 4.4223x; 1.0004x over previous
import jax
import jax.numpy as jnp
from jax import lax
from jax.experimental import pallas as pl
from jax.experimental.pallas import tpu as pltpu
from jax.experimental.pallas import tpu_sc as plsc

N = 10000
NREL = 474
E = 160000
H = 256

NC = 2
NS = 16
NW = NC * NS
C = 112
NCH = 45
EPW = C * NCH
EP = EPW * NW
NROW = EP // C
ACC_R = 10240
RPT = ACC_R // NS
QH = 32
NQ = H // QH
APAD = 10240
BN = 512
NEGB = -1000.0


def _lanes():
    return lax.iota(jnp.int32, 16)


def _lg(x, idx):
    dnums = lax.GatherDimensionNumbers(
        offset_dims=(), collapsed_slice_dims=(0,), start_index_map=(0,))
    return lax.gather(x, idx[:, None], dnums, (1,),
                      mode=lax.GatherScatterMode.PROMISE_IN_BOUNDS)


def _hsum(x):
    lane = _lanes()
    for d in (1, 2, 4, 8):
        x = x + _lg(x, lane ^ d)
    return x


def _k1_body(ent_h, rel_h, src_h, dstg_h, dsts_h, rid_h,
             n0_h, n1_h, n2_h, tacc_h,
             idx_s, idx_dg, idx_ds, idx_r, rows_s, rows_d, rows_r,
             nbuf, trow, zbuf16, tacc, sem0, sem1, sem2, sem3):
    cid = lax.axis_index("c")
    sid = lax.axis_index("s")
    wid = cid * NS + sid
    lane = _lanes()

    def zb(i, _):
        zbuf16[i, pl.ds(0, 16)] = jnp.zeros((16,), jnp.float32)
        return 0
    lax.fori_loop(0, RPT // 8, zb, 0)

    def zc(k, _):
        pltpu.sync_copy(zbuf16, tacc.at[pl.ds(sid * RPT + k * (RPT // 8),
                                              RPT // 8)])
        return 0
    lax.fori_loop(0, 8, zc, 0)
    plsc.subcore_barrier()

    pltpu.sync_copy(src_h.at[wid], idx_s)
    pltpu.sync_copy(dstg_h.at[wid], idx_dg)
    pltpu.sync_copy(dsts_h.at[wid], idx_ds)
    pltpu.sync_copy(rid_h.at[wid], idx_r)

    def chunk(ch, _):
        r = wid * NCH + ch
        ca = pltpu.async_copy(ent_h.at[idx_s.at[ch]],
                              rows_s, sem0)
        cb = pltpu.async_copy(ent_h.at[idx_dg.at[ch]],
                              rows_d, sem1)
        cc = pltpu.async_copy(rel_h.at[idx_r.at[ch]],
                              rows_r, sem2)
        ca.wait()
        cb.wait()
        cc.wait()

        def group(g, _):
            def edge(j16, carry):
                vn, vc, ve = carry
                j = g * 16 + j16
                accn = jnp.zeros((16,), jnp.float32)
                accc = jnp.zeros((16,), jnp.float32)
                acce = jnp.zeros((16,), jnp.float32)
                for hv in range(16):
                    s4 = rows_s[j, pl.ds(hv * 16, 16)]
                    d4 = rows_d[j, pl.ds(hv * 16, 16)]
                    r4 = rows_r[j, pl.ds(hv * 16, 16)]
                    accn = accn + s4 * d4
                    accc = accc + (s4 * r4) * d4
                    acce = acce + r4 * d4
                hn = _hsum(accn)
                hc = _hsum(accc)
                he = _hsum(acce)
                vn = jnp.where(lane == j16, hn, vn)
                vc = jnp.where(lane == j16, hc, vc)
                ve = jnp.where(lane == j16, he, ve)
                nsel = jnp.where(lane < 5, hn,
                                 jnp.where(lane < 10, hc, he))
                sub5 = jnp.where(lane < 5, 0,
                                 jnp.where(lane < 10, 5, 10))
                k = lane - sub5
                shift = jnp.where(k == 0, 0.0,
                                  jnp.where(k == 1, 600.0,
                                            jnp.where(k == 2, 1200.0,
                                                      jnp.where(k == 3,
                                                                1800.0,
                                                                2400.0))))
                x = (nsel - shift) * 0.25
                trow[j, pl.ds(0, 16)] = jnp.exp(jnp.minimum(x, 74.0))
                return vn, vc, ve

            zero = jnp.zeros((16,), jnp.float32)
            vn, vc, ve = lax.fori_loop(0, 16, edge, (zero, zero, zero))
            nbuf[0, pl.ds(g * 16, 16)] = vn
            nbuf[1, pl.ds(g * 16, 16)] = vc
            nbuf[2, pl.ds(g * 16, 16)] = ve
            return 0

        lax.fori_loop(0, C // 16, group, 0)
        pltpu.sync_copy(nbuf.at[0], n0_h.at[r])
        pltpu.sync_copy(nbuf.at[1], n1_h.at[r])
        pltpu.sync_copy(nbuf.at[2], n2_h.at[r])
        dd = pltpu.async_copy(trow, tacc.at[idx_ds.at[ch]], sem3, add=True)
        dd.wait()
        return 0

    lax.fori_loop(0, NCH, chunk, 0)
    plsc.subcore_barrier()
    pltpu.sync_copy(tacc.at[pl.ds(sid * RPT, RPT)],
                    tacc_h.at[cid, sid])


def _run_k1(ent, rel, src2, dstg2, dsts2, rid2):
    mesh = plsc.VectorSubcoreMesh(core_axis_name="c", subcore_axis_name="s")
    out_type = (
        jax.ShapeDtypeStruct((NROW, C), jnp.float32),
        jax.ShapeDtypeStruct((NROW, C), jnp.float32),
        jax.ShapeDtypeStruct((NROW, C), jnp.float32),
        jax.ShapeDtypeStruct((NC, NS, RPT, 16), jnp.float32),
    )
    scratch = [
        pltpu.VMEM((NCH, C), jnp.int32),
        pltpu.VMEM((NCH, C), jnp.int32),
        pltpu.VMEM((NCH, C), jnp.int32),
        pltpu.VMEM((NCH, C), jnp.int32),
        pltpu.VMEM((C, H), jnp.float32),
        pltpu.VMEM((C, H), jnp.float32),
        pltpu.VMEM((C, H), jnp.float32),
        pltpu.VMEM((3, C), jnp.float32),
        pltpu.VMEM((C, 16), jnp.float32),
        pltpu.VMEM((RPT // 8, 16), jnp.float32),
        pltpu.VMEM_SHARED((ACC_R, 16), jnp.float32),
        pltpu.SemaphoreType.DMA,
        pltpu.SemaphoreType.DMA,
        pltpu.SemaphoreType.DMA,
        pltpu.SemaphoreType.DMA,
    ]
    f = pl.kernel(_k1_body, out_type=out_type, mesh=mesh,
                  scratch_types=scratch,
                  compiler_params=pltpu.CompilerParams(
                      use_tc_tiling_on_sc=False))
    return f(ent, rel, src2, dstg2, dsts2, rid2)


def _k2_body(t_ref, o_ref):
    t = t_ref[0] + t_ref[1]
    cols = []
    for sub in range(3):
        cands = []
        for k in range(5):
            tk = t[:, sub * 5 + k]
            cands.append(jnp.where(tk > 0.0,
                                   600.0 * k + 4.0 * jnp.log(tk), -3.0e38))
        mh = cands[0]
        for c in cands[1:]:
            mh = jnp.maximum(mh, c)
        cols.append(jnp.where(mh < -1.0e38, 0.0, mh))
    out = jnp.stack(cols, axis=-1)
    o_ref[...] = jnp.concatenate(
        [out, jnp.zeros((out.shape[0], 13), jnp.float32)], axis=-1)


def _run_k2(tacc):
    return pl.pallas_call(
        _k2_body,
        out_shape=jax.ShapeDtypeStruct((ACC_R, 16), jnp.float32),
    )(tacc)


def _k4_body(*refs):
    ent_qs = refs[0:NQ]
    rel_qs = refs[NQ:2 * NQ]
    (src_h, dsts_h, rid_h, n0_h, n1_h, n2_h, mhat_h,
     sacc_h, u_h, ebuf_h,
     idx_s, idx_ds, idx_r, nbuf, mrow, erow,
     rows_s, rows_r, stg_n, stg_c, stg_e, zbuf16, zbuf64,
     acc_n, acc_c, acc_e, sacc,
     sem0, sem1, sem2, sem3, sem4, sem5, sem6) = refs[2 * NQ:]
    cid = lax.axis_index("c")
    sid = lax.axis_index("s")
    wid = cid * NS + sid
    lane = _lanes()
    pltpu.sync_copy(src_h.at[wid], idx_s)
    pltpu.sync_copy(dsts_h.at[wid], idx_ds)
    pltpu.sync_copy(rid_h.at[wid], idx_r)

    def zb(i, _):
        zbuf16[i, pl.ds(0, 16)] = jnp.zeros((16,), jnp.float32)
        return 0
    lax.fori_loop(0, RPT // 8, zb, 0)

    def zb2(i, _):
        for hv in range(QH // 16):
            zbuf64[i, pl.ds(hv * 16, 16)] = jnp.zeros((16,), jnp.float32)
        return 0
    lax.fori_loop(0, RPT // 8, zb2, 0)

    def zs(k, _):
        pltpu.sync_copy(zbuf16, sacc.at[pl.ds(sid * RPT + k * (RPT // 8),
                                              RPT // 8)])
        return 0
    lax.fori_loop(0, 8, zs, 0)

    for q in range(NQ):
        def za(k, _):
            b = sid * RPT + k * (RPT // 8)
            pltpu.sync_copy(zbuf64, acc_n.at[pl.ds(b, RPT // 8)])
            pltpu.sync_copy(zbuf64, acc_c.at[pl.ds(b, RPT // 8)])
            pltpu.sync_copy(zbuf64, acc_e.at[pl.ds(b, RPT // 8)])
            return 0
        lax.fori_loop(0, 8, za, 0)
        plsc.subcore_barrier()

        def chunk(ch, _):
            r = wid * NCH + ch
            ca = pltpu.async_copy(ent_qs[q].at[idx_s.at[ch]],
                                  rows_s, sem0)
            cb = pltpu.async_copy(rel_qs[q].at[idx_r.at[ch]],
                                  rows_r, sem1)
            if q == 0:
                pltpu.sync_copy(n0_h.at[r], nbuf.at[0])
                pltpu.sync_copy(n1_h.at[r], nbuf.at[1])
                pltpu.sync_copy(n2_h.at[r], nbuf.at[2])
                cm = pltpu.async_copy(mhat_h.at[idx_ds.at[ch]], mrow, sem2)
                cm.wait()
            else:
                ce = pltpu.async_copy(ebuf_h.at[r], erow, sem2)
                ce.wait()
            ca.wait()
            cb.wait()

            if q == 0:
                def egroup(g, _):
                    n0v = nbuf[0, pl.ds(g * 16, 16)]
                    n1v = nbuf[1, pl.ds(g * 16, 16)]
                    n2v = nbuf[2, pl.ds(g * 16, 16)]
                    for j16 in range(16):
                        j = g * 16 + j16
                        nv = jnp.where(lane == 0, n0v[j16],
                                       jnp.where(lane == 1, n1v[j16],
                                                 jnp.where(lane == 2,
                                                           n2v[j16], NEGB)))
                        mj = mrow[j, pl.ds(0, 16)]
                        erow[j, pl.ds(0, 16)] = jnp.exp(nv - mj)
                    return 0
                lax.fori_loop(0, C // 16, egroup, 0)
                ds1 = pltpu.async_copy(erow, sacc.at[idx_ds.at[ch]],
                                       sem3, add=True)
                ds2 = pltpu.async_copy(erow, ebuf_h.at[r], sem2)
                ds1.wait()
                ds2.wait()

            def mgroup(g, _):
                for j16 in range(16):
                    j = g * 16 + j16
                    ev = erow[j, pl.ds(0, 16)]
                    en = ev[0]
                    ec = ev[1]
                    ee = ev[2]
                    for hv in range(QH // 16):
                        s4 = rows_s[j, pl.ds(hv * 16, 16)]
                        r4 = rows_r[j, pl.ds(hv * 16, 16)]
                        stg_n[j, pl.ds(hv * 16, 16)] = s4 * en
                        stg_c[j, pl.ds(hv * 16, 16)] = (s4 * r4) * ec
                        stg_e[j, pl.ds(hv * 16, 16)] = r4 * ee
                return 0
            lax.fori_loop(0, C // 16, mgroup, 0)

            ids = idx_ds.at[ch]
            da = pltpu.async_copy(stg_n, acc_n.at[ids], sem4, add=True)
            db = pltpu.async_copy(stg_c, acc_c.at[ids], sem5, add=True)
            dc = pltpu.async_copy(stg_e, acc_e.at[ids], sem6, add=True)
            da.wait()
            db.wait()
            dc.wait()
            return 0

        lax.fori_loop(0, NCH, chunk, 0)
        plsc.subcore_barrier()
        base = sid * RPT
        pltpu.sync_copy(acc_n.at[pl.ds(base, RPT)], u_h.at[q, cid, 0, sid])
        pltpu.sync_copy(acc_c.at[pl.ds(base, RPT)], u_h.at[q, cid, 1, sid])
        pltpu.sync_copy(acc_e.at[pl.ds(base, RPT)], u_h.at[q, cid, 2, sid])
        plsc.subcore_barrier()

    pltpu.sync_copy(sacc.at[pl.ds(sid * RPT, RPT)],
                    sacc_h.at[cid, sid])


def _run_k4(ent_qs, rel_qs, src2, dsts2, rid2, n0, n1, n2, mhat):
    mesh = plsc.VectorSubcoreMesh(core_axis_name="c", subcore_axis_name="s")
    out_type = (
        jax.ShapeDtypeStruct((NC, NS, RPT, 16), jnp.float32),
        jax.ShapeDtypeStruct((NQ, NC, 3, NS, RPT, QH), jnp.float32),
        jax.ShapeDtypeStruct((NROW, C, 16), jnp.float32),
    )
    scratch = [
        pltpu.VMEM((NCH, C), jnp.int32),
        pltpu.VMEM((NCH, C), jnp.int32),
        pltpu.VMEM((NCH, C), jnp.int32),
        pltpu.VMEM((3, C), jnp.float32),
        pltpu.VMEM((C, 16), jnp.float32),
        pltpu.VMEM((C, 16), jnp.float32),
        pltpu.VMEM((C, QH), jnp.float32),
        pltpu.VMEM((C, QH), jnp.float32),
        pltpu.VMEM((C, QH), jnp.float32),
        pltpu.VMEM((C, QH), jnp.float32),
        pltpu.VMEM((C, QH), jnp.float32),
        pltpu.VMEM((RPT // 8, 16), jnp.float32),
        pltpu.VMEM((RPT // 8, QH), jnp.float32),
        pltpu.VMEM_SHARED((ACC_R, QH), jnp.float32),
        pltpu.VMEM_SHARED((ACC_R, QH), jnp.float32),
        pltpu.VMEM_SHARED((ACC_R, QH), jnp.float32),
        pltpu.VMEM_SHARED((ACC_R, 16), jnp.float32),
        pltpu.SemaphoreType.DMA,
        pltpu.SemaphoreType.DMA,
        pltpu.SemaphoreType.DMA,
        pltpu.SemaphoreType.DMA,
        pltpu.SemaphoreType.DMA,
        pltpu.SemaphoreType.DMA,
        pltpu.SemaphoreType.DMA,
    ]
    f = pl.kernel(_k4_body, out_type=out_type, mesh=mesh,
                  scratch_types=scratch,
                  compiler_params=pltpu.CompilerParams(
                      use_tc_tiling_on_sc=False))
    return f(*ent_qs, *rel_qs, src2, dsts2, rid2, n0, n1, n2, mhat)


def _k5_body(u_ref, s_ref, ent_ref, wn_ref, wc_ref, we_ref, o_ref):
    sp = s_ref[0] + s_ref[1]
    acc = ent_ref[...]
    for sub, w_ref in ((0, wn_ref), (1, wc_ref), (2, we_ref)):
        u = u_ref[:, 0, sub] + u_ref[:, 1, sub]
        neigh = jnp.concatenate([u[q] for q in range(NQ)], axis=-1)
        neigh = neigh / (sp[:, sub][:, None] + 1e-16)
        z = jnp.dot(neigh, w_ref[...], preferred_element_type=jnp.float32)
        acc = acc + jnp.tanh(z)
    o_ref[...] = acc


def _run_k5(u, sacc, ent, wn, wc, we):
    grid = APAD // BN
    return pl.pallas_call(
        _k5_body,
        grid=(grid,),
        in_specs=[
            pl.BlockSpec((NQ, NC, 3, BN, QH), lambda i: (0, 0, 0, i, 0)),
            pl.BlockSpec((NC, BN, 16), lambda i: (0, i, 0)),
            pl.BlockSpec((BN, H), lambda i: (i, 0)),
            pl.BlockSpec((H, H), lambda i: (0, 0)),
            pl.BlockSpec((H, H), lambda i: (0, 0)),
            pl.BlockSpec((H, H), lambda i: (0, 0)),
        ],
        out_specs=pl.BlockSpec((BN, H), lambda i: (i, 0)),
        out_shape=jax.ShapeDtypeStruct((APAD, H), jnp.float32),
    )(u, sacc, ent, wn, wc, we)


def kernel(edge_index, rel_id, ent_emb, rel_emb_0, rel_emb_1,
           W_edge_0, W_node_0, W_comp_0, W_edge_1, W_node_1, W_comp_1):
    src = edge_index[0]
    dst = edge_index[1]
    pad = EP - E
    srcp = jnp.concatenate([src, jnp.zeros((pad,), jnp.int32)])
    ridp = jnp.concatenate([rel_id, jnp.zeros((pad,), jnp.int32)])
    dstg = jnp.concatenate([dst, jnp.zeros((pad,), jnp.int32)])
    dsts = jnp.concatenate([dst, jnp.full((pad,), N, jnp.int32)])
    order = jnp.argsort(dsts)
    deal = lambda a: a[order].reshape(C, NROW).T.reshape(NW, NCH, C).copy()
    src2 = deal(srcp)
    rid2 = deal(ridp)
    dstg2 = deal(dstg)
    dsts2 = deal(dsts)

    ent = jnp.pad(ent_emb, ((0, APAD - N), (0, 0)))
    layers = (
        (rel_emb_0, W_edge_0, W_node_0, W_comp_0),
        (rel_emb_1, W_edge_1, W_node_1, W_comp_1),
    )
    for rel, We, Wn, Wc in layers:
        n0, n1, n2, tacc = _run_k1(ent, rel, src2, dstg2, dsts2, rid2)
        mhat = _run_k2(tacc.reshape(NC, ACC_R, 16))
        ent_qs = tuple(ent[:, q * QH:(q + 1) * QH] for q in range(NQ))
        rel_qs = tuple(rel[:, q * QH:(q + 1) * QH] for q in range(NQ))
        sacc, u, _ = _run_k4(ent_qs, rel_qs, src2, dsts2, rid2,
                             n0, n1, n2, mhat)
        ent = _run_k5(u.reshape(NQ, NC, 3, ACC_R, QH),
                      sacc.reshape(NC, ACC_R, 16), ent, Wn, Wc, We)
    return ent[:N]
